# Initial kernel scaffold; baseline (speedup 1.0000x reference)
import jax
import jax.numpy as jnp
from jax.experimental import pallas as pl


def kernel(node_feats, edge_feats, edge_index, W_nn, b_nn, W_en, b_en, W_un, b_un, W_l, b_l, W_r, b_r, W_ee, b_ee, W_ue, b_ue):
    raise NotImplementedError("write your pallas kernel here")


# speedup vs baseline: 3.5324x; 3.5324x over previous
import functools

import jax
import jax.numpy as jnp
from jax import lax
from jax.experimental import pallas as pl
from jax.experimental.pallas import tpu as pltpu
from jax.experimental.pallas import tpu_sc as plsc

V = 10000
E = 320000
NIF = 128
EIF = 16
H = 50
HP = 64
NC = 2
NS = 16
NW = NC * NS
EW = E // NW
CH = 80
NCHUNK = EW // CH
VP = 10240
VSUB = VP // NS


def _node_prep(nf, Wl, bl, Wr, br, Wnn, bnn):
    def body(nf_ref, Wl_ref, bl_ref, Wr_ref, br_ref, Wnn_ref, bnn_ref,
             T_ref, nnf_ref):
        x = nf_ref[...]
        left = jnp.dot(x, Wl_ref[...], preferred_element_type=jnp.float32) + bl_ref[...]
        right = jnp.dot(x, Wr_ref[...], preferred_element_type=jnp.float32) + br_ref[...]
        T_ref[...] = jnp.concatenate([left, right], axis=1)
        nnf = jnp.dot(x, Wnn_ref[...], preferred_element_type=jnp.float32) + bnn_ref[...]
        nnf_ref[...] = jnp.maximum(nnf, 0.0)

    return pl.pallas_call(
        body,
        out_shape=[jax.ShapeDtypeStruct((V, 2 * HP), jnp.float32),
                   jax.ShapeDtypeStruct((V, HP), jnp.float32)],
    )(nf, Wl, bl, Wr, br, Wnn, bnn)


def _edge_prep(ef, Wen, ben):
    TE = 4000

    def body(ef_ref, W_ref, b_ref, out_ref):
        y = jnp.dot(ef_ref[...], W_ref[...], preferred_element_type=jnp.float32)
        out_ref[...] = jnp.maximum(y + b_ref[...], 0.0)

    return pl.pallas_call(
        body,
        grid=(E // TE,),
        in_specs=[pl.BlockSpec((TE, EIF), lambda i: (i, 0)),
                  pl.BlockSpec((EIF, HP), lambda i: (0, 0)),
                  pl.BlockSpec((1, HP), lambda i: (0, 0))],
        out_specs=pl.BlockSpec((TE, HP), lambda i: (i, 0)),
        out_shape=jax.ShapeDtypeStruct((E, HP), jnp.float32),
    )(ef, Wen, ben)


def _sc_gather_scatter(T, src, dst, e2n, zrows):
    mesh = plsc.VectorSubcoreMesh(core_axis_name="c", subcore_axis_name="s")

    @functools.partial(
        pl.kernel,
        mesh=mesh,
        out_type=[jax.ShapeDtypeStruct((E, 2 * HP), jnp.float32),
                  jax.ShapeDtypeStruct((E, 2 * HP), jnp.float32),
                  jax.ShapeDtypeStruct((2 * VP, HP), jnp.float32)],
        scratch_types=[pltpu.VMEM((CH,), jnp.int32),
                       pltpu.VMEM((CH,), jnp.int32),
                       pltpu.VMEM((CH, 2 * HP), jnp.float32),
                       pltpu.VMEM((CH, 2 * HP), jnp.float32),
                       pltpu.VMEM((CH, HP), jnp.float32),
                       pltpu.VMEM_SHARED((VP, HP), jnp.float32),
                       pltpu.SemaphoreType.DMA,
                       pltpu.SemaphoreType.DMA],
    )
    def k(T_hbm, src_hbm, dst_hbm, e2n_hbm, z_hbm, gs_out, gd_out, enf_out,
          sidx, didx, gs_v, gd_v, ev, acc, sem0, sem1):
        cid = lax.axis_index("c")
        sid = lax.axis_index("s")
        wid = sid * NC + cid
        pltpu.sync_copy(z_hbm.at[pl.ds(sid * VSUB, VSUB)],
                        acc.at[pl.ds(sid * VSUB, VSUB)])
        plsc.subcore_barrier()

        def chunk(i, carry):
            base = wid * EW + i * CH
            pltpu.sync_copy(src_hbm.at[pl.ds(base, CH)], sidx)
            pltpu.sync_copy(dst_hbm.at[pl.ds(base, CH)], didx)
            cps = pltpu.async_copy(T_hbm.at[sidx], gs_v, sem0)
            cpd = pltpu.async_copy(T_hbm.at[didx], gd_v, sem1)
            pltpu.sync_copy(e2n_hbm.at[pl.ds(base, CH)], ev)
            pltpu.sync_copy(ev, acc.at[didx], add=True)
            cps.wait()
            cpd.wait()
            pltpu.sync_copy(gs_v, gs_out.at[pl.ds(base, CH)])
            pltpu.sync_copy(gd_v, gd_out.at[pl.ds(base, CH)])
            return carry

        lax.fori_loop(0, NCHUNK, chunk, 0)
        plsc.subcore_barrier()
        pltpu.sync_copy(acc.at[pl.ds(sid * VSUB, VSUB)],
                        enf_out.at[pl.ds(cid * VP + sid * VSUB, VSUB)])

    return k(T, src, dst, e2n, zrows)


def _edge_final(gs, gd, ef, W1, W2, W3, Wee, bee, bue):
    TE = 4000

    def body(gs_ref, gd_ref, ef_ref, W1_ref, W2_ref, W3_ref, Wee_ref,
             bee_ref, bue_ref, out_ref):
        g_s = gs_ref[...]
        g_d = gd_ref[...]
        first = jnp.maximum(g_s[:, :HP] + g_d[:, HP:], 0.0)
        second = jnp.maximum(g_s[:, HP:] + g_d[:, :HP], 0.0)
        third = jnp.dot(ef_ref[...], Wee_ref[...], preferred_element_type=jnp.float32)
        third = jnp.maximum(third + bee_ref[...], 0.0)
        acc = jnp.dot(first, W1_ref[...], preferred_element_type=jnp.float32)
        acc += jnp.dot(second, W2_ref[...], preferred_element_type=jnp.float32)
        acc += jnp.dot(third, W3_ref[...], preferred_element_type=jnp.float32)
        out_ref[...] = jnp.maximum(acc + bue_ref[...], 0.0)

    return pl.pallas_call(
        body,
        grid=(E // TE,),
        in_specs=[pl.BlockSpec((TE, 2 * HP), lambda i: (i, 0)),
                  pl.BlockSpec((TE, 2 * HP), lambda i: (i, 0)),
                  pl.BlockSpec((TE, EIF), lambda i: (i, 0)),
                  pl.BlockSpec((HP, HP), lambda i: (0, 0)),
                  pl.BlockSpec((HP, HP), lambda i: (0, 0)),
                  pl.BlockSpec((HP, HP), lambda i: (0, 0)),
                  pl.BlockSpec((EIF, HP), lambda i: (0, 0)),
                  pl.BlockSpec((1, HP), lambda i: (0, 0)),
                  pl.BlockSpec((1, HP), lambda i: (0, 0))],
        out_specs=pl.BlockSpec((TE, HP), lambda i: (i, 0)),
        out_shape=jax.ShapeDtypeStruct((E, HP), jnp.float32),
    )(gs, gd, ef, W1, W2, W3, Wee, bee, bue)


def _node_final(nnf, enf2, Wt, Wb, bun):
    def body(nnf_ref, enf_ref, Wt_ref, Wb_ref, b_ref, out_ref):
        e = enf_ref[:V, :] + enf_ref[VP:VP + V, :]
        acc = jnp.dot(nnf_ref[...], Wt_ref[...], preferred_element_type=jnp.float32)
        acc += jnp.dot(e, Wb_ref[...], preferred_element_type=jnp.float32)
        out_ref[...] = jnp.maximum(acc + b_ref[...], 0.0)

    return pl.pallas_call(
        body,
        out_shape=jax.ShapeDtypeStruct((V, HP), jnp.float32),
    )(nnf, enf2, Wt, Wb, bun)


def _pad_w(W, rows, cols=HP):
    out = jnp.zeros((rows, cols), jnp.float32)
    return out.at[:W.shape[0], :W.shape[1]].set(W)


def _pad_b(b):
    return jnp.zeros((1, HP), jnp.float32).at[0, :b.shape[0]].set(b)


def kernel(node_feats, edge_feats, edge_index, W_nn, b_nn, W_en, b_en,
           W_un, b_un, W_l, b_l, W_r, b_r, W_ee, b_ee, W_ue, b_ue):
    src = edge_index[0].astype(jnp.int32)
    dst = edge_index[1].astype(jnp.int32)

    Wl = _pad_w(W_l, NIF)
    Wr = _pad_w(W_r, NIF)
    Wnn = _pad_w(W_nn, NIF)
    Wen = _pad_w(W_en, EIF)
    Wee = _pad_w(W_ee, EIF)
    W1 = _pad_w(W_ue[:H], HP)
    W2 = _pad_w(W_ue[H:2 * H], HP)
    W3 = _pad_w(W_ue[2 * H:], HP)
    Wt = _pad_w(W_un[:H], HP)
    Wb = _pad_w(W_un[H:], HP)

    T, nnf = _node_prep(node_feats, Wl, _pad_b(b_l), Wr, _pad_b(b_r),
                        Wnn, _pad_b(b_nn))
    e2n = _edge_prep(edge_feats, Wen, _pad_b(b_en))
    zrows = jnp.zeros((VP, HP), jnp.float32)
    gs, gd, enf2 = _sc_gather_scatter(T, src, dst, e2n, zrows)
    new_edge = _edge_final(gs, gd, edge_feats, W1, W2, W3, Wee,
                           _pad_b(b_ee), _pad_b(b_ue))
    new_node = _node_final(nnf, enf2, Wt, Wb, _pad_b(b_un))
    return new_node[:, :H], new_edge[:, :H]


# speedup vs baseline: 4.2967x; 1.2164x over previous
import functools

import jax
import jax.numpy as jnp
from jax import lax
from jax.experimental import pallas as pl
from jax.experimental.pallas import tpu as pltpu
from jax.experimental.pallas import tpu_sc as plsc

V = 10000
E = 320000
NIF = 128
EIF = 16
H = 50
HP = 64
NC = 2
NS = 16
NW = NC * NS
EW = E // NW
CH = 40
NCHUNK = EW // CH
VP = 10240
VSUB = VP // NS


def _node_prep(nf, Wl, bl, Wr, br, Wnn, bnn):
    def body(nf_ref, Wl_ref, bl_ref, Wr_ref, br_ref, Wnn_ref, bnn_ref,
             T_ref, nnf_ref):
        x = nf_ref[...]
        left = jnp.dot(x, Wl_ref[...], preferred_element_type=jnp.float32) + bl_ref[...]
        right = jnp.dot(x, Wr_ref[...], preferred_element_type=jnp.float32) + br_ref[...]
        T_ref[...] = jnp.concatenate([left, right], axis=1)
        nnf = jnp.dot(x, Wnn_ref[...], preferred_element_type=jnp.float32) + bnn_ref[...]
        nnf_ref[...] = jnp.maximum(nnf, 0.0)

    return pl.pallas_call(
        body,
        out_shape=[jax.ShapeDtypeStruct((V, 2 * HP), jnp.float32),
                   jax.ShapeDtypeStruct((V, HP), jnp.float32)],
    )(nf, Wl, bl, Wr, br, Wnn, bnn)


def _edge_prep(ef, Wen, ben):
    TE = 4000

    def body(ef_ref, W_ref, b_ref, out_ref):
        y = jnp.dot(ef_ref[...], W_ref[...], preferred_element_type=jnp.float32)
        out_ref[...] = jnp.maximum(y + b_ref[...], 0.0)

    return pl.pallas_call(
        body,
        grid=(E // TE,),
        in_specs=[pl.BlockSpec((TE, EIF), lambda i: (i, 0)),
                  pl.BlockSpec((EIF, HP), lambda i: (0, 0)),
                  pl.BlockSpec((1, HP), lambda i: (0, 0))],
        out_specs=pl.BlockSpec((TE, HP), lambda i: (i, 0)),
        out_shape=jax.ShapeDtypeStruct((E, HP), jnp.float32),
    )(ef, Wen, ben)


def _sc_gather_scatter(T, src3, dst3, e2n, zrows):
    mesh = plsc.VectorSubcoreMesh(core_axis_name="c", subcore_axis_name="s")

    @functools.partial(
        pl.kernel,
        mesh=mesh,
        out_type=[jax.ShapeDtypeStruct((E, 2 * HP), jnp.float32),
                  jax.ShapeDtypeStruct((E, 2 * HP), jnp.float32),
                  jax.ShapeDtypeStruct((2 * VP, HP), jnp.float32)],
        scratch_types=[pltpu.VMEM((CH,), jnp.int32),
                       pltpu.VMEM((CH,), jnp.int32),
                       pltpu.VMEM((CH,), jnp.int32),
                       pltpu.VMEM((CH,), jnp.int32),
                       pltpu.VMEM((CH,), jnp.int32),
                       pltpu.VMEM((CH,), jnp.int32),
                       pltpu.VMEM((CH,), jnp.int32),
                       pltpu.VMEM((CH,), jnp.int32),
                       pltpu.VMEM((CH, 2 * HP), jnp.float32),
                       pltpu.VMEM((CH, 2 * HP), jnp.float32),
                       pltpu.VMEM((CH, 2 * HP), jnp.float32),
                       pltpu.VMEM((CH, 2 * HP), jnp.float32),
                       pltpu.VMEM((CH, HP), jnp.float32),
                       pltpu.VMEM((CH, HP), jnp.float32),
                       pltpu.VMEM_SHARED((VP, HP), jnp.float32),
                       pltpu.SemaphoreType.DMA,
                       pltpu.SemaphoreType.DMA,
                       pltpu.SemaphoreType.DMA,
                       pltpu.SemaphoreType.DMA,
                       pltpu.SemaphoreType.DMA,
                       pltpu.SemaphoreType.DMA,
                       pltpu.SemaphoreType.DMA,
                       pltpu.SemaphoreType.DMA],
    )
    def k(T_hbm, src_hbm, dst_hbm, e2n_hbm, z_hbm, gs_out, gd_out, enf_out,
          si0, si1, si2, si3, di0, di1, di2, di3,
          gs_v0, gs_v1, gd_v0, gd_v1, ev0, ev1, acc,
          gsem0, gsem1, wsem0, wsem1, isem0, isem1, isem2, isem3):
        cid = lax.axis_index("c")
        sid = lax.axis_index("s")
        wid = sid * NC + cid
        sis = (si0, si1, si2, si3)
        dis = (di0, di1, di2, di3)
        gs_v = (gs_v0, gs_v1)
        gd_v = (gd_v0, gd_v1)
        ev = (ev0, ev1)
        gsems = (gsem0, gsem1)
        wsems = (wsem0, wsem1)
        isems = (isem0, isem1, isem2, isem3)
        pltpu.sync_copy(z_hbm.at[pl.ds(sid * VSUB, VSUB)],
                        acc.at[pl.ds(sid * VSUB, VSUB)])
        plsc.subcore_barrier()

        def i_start(i, s):
            pltpu.async_copy(src_hbm.at[wid, i], sis[s], isems[s])
            pltpu.async_copy(dst_hbm.at[wid, i], dis[s], isems[s])

        def i_wait(i, s):
            pltpu.make_async_copy(src_hbm.at[wid, i], sis[s], isems[s]).wait()
            pltpu.make_async_copy(dst_hbm.at[wid, i], dis[s], isems[s]).wait()

        def g_start(i, s, b):
            base = wid * EW + i * CH
            pltpu.async_copy(T_hbm.at[sis[s]], gs_v[b], gsems[b])
            pltpu.async_copy(T_hbm.at[dis[s]], gd_v[b], gsems[b])
            pltpu.async_copy(e2n_hbm.at[pl.ds(base, CH)], ev[b], gsems[b])

        def g_wait(i, s, b):
            base = wid * EW + i * CH
            pltpu.make_async_copy(T_hbm.at[sis[s]], gs_v[b], gsems[b]).wait()
            pltpu.make_async_copy(T_hbm.at[dis[s]], gd_v[b], gsems[b]).wait()
            pltpu.make_async_copy(e2n_hbm.at[pl.ds(base, CH)], ev[b], gsems[b]).wait()

        def w_start(i, s, b):
            base = wid * EW + i * CH
            pltpu.async_copy(gs_v[b], gs_out.at[pl.ds(base, CH)], wsems[b])
            pltpu.async_copy(gd_v[b], gd_out.at[pl.ds(base, CH)], wsems[b])
            pltpu.sync_copy(ev[b], acc.at[dis[s]], add=True)

        def w_wait(i, b):
            base = wid * EW + i * CH
            pltpu.make_async_copy(gs_v[b], gs_out.at[pl.ds(base, CH)], wsems[b]).wait()
            pltpu.make_async_copy(gd_v[b], gd_out.at[pl.ds(base, CH)], wsems[b]).wait()

        for s in range(4):
            i_start(s, s)
        i_wait(0, 0)
        i_wait(1, 1)
        g_start(0, 0, 0)
        g_start(1, 1, 1)

        def body(g, carry):
            for b4 in range(4):
                j = 4 * g + b4
                b = b4 % 2
                g_wait(j, b4, b)
                w_start(j, b4, b)
                w_wait(j, b)
                i_wait(j + 2, (b4 + 2) % 4)
                g_start(j + 2, (b4 + 2) % 4, b)
                i_start(j + 4, b4)
            return carry

        lax.fori_loop(0, (NCHUNK - 2) // 4, body, 0)
        for t in range(2):
            j = NCHUNK - 2 + t
            s = j % 4
            b = j % 2
            g_wait(j, s, b)
            w_start(j, s, b)
            w_wait(j, b)
        i_wait(NCHUNK, NCHUNK % 4)
        i_wait(NCHUNK + 1, (NCHUNK + 1) % 4)
        plsc.subcore_barrier()
        pltpu.sync_copy(acc.at[pl.ds(sid * VSUB, VSUB)],
                        enf_out.at[pl.ds(cid * VP + sid * VSUB, VSUB)])

    return k(T, src3, dst3, e2n, zrows)


def _edge_final(gs, gd, ef, W1, W2, W3, Wee, bee, bue):
    TE = 4000

    def body(gs_ref, gd_ref, ef_ref, W1_ref, W2_ref, W3_ref, Wee_ref,
             bee_ref, bue_ref, out_ref):
        g_s = gs_ref[...]
        g_d = gd_ref[...]
        first = jnp.maximum(g_s[:, :HP] + g_d[:, HP:], 0.0)
        second = jnp.maximum(g_s[:, HP:] + g_d[:, :HP], 0.0)
        third = jnp.dot(ef_ref[...], Wee_ref[...], preferred_element_type=jnp.float32)
        third = jnp.maximum(third + bee_ref[...], 0.0)
        acc = jnp.dot(first, W1_ref[...], preferred_element_type=jnp.float32)
        acc += jnp.dot(second, W2_ref[...], preferred_element_type=jnp.float32)
        acc += jnp.dot(third, W3_ref[...], preferred_element_type=jnp.float32)
        out_ref[...] = jnp.maximum(acc + bue_ref[...], 0.0)

    return pl.pallas_call(
        body,
        grid=(E // TE,),
        in_specs=[pl.BlockSpec((TE, 2 * HP), lambda i: (i, 0)),
                  pl.BlockSpec((TE, 2 * HP), lambda i: (i, 0)),
                  pl.BlockSpec((TE, EIF), lambda i: (i, 0)),
                  pl.BlockSpec((HP, HP), lambda i: (0, 0)),
                  pl.BlockSpec((HP, HP), lambda i: (0, 0)),
                  pl.BlockSpec((HP, HP), lambda i: (0, 0)),
                  pl.BlockSpec((EIF, HP), lambda i: (0, 0)),
                  pl.BlockSpec((1, HP), lambda i: (0, 0)),
                  pl.BlockSpec((1, HP), lambda i: (0, 0))],
        out_specs=pl.BlockSpec((TE, HP), lambda i: (i, 0)),
        out_shape=jax.ShapeDtypeStruct((E, HP), jnp.float32),
    )(gs, gd, ef, W1, W2, W3, Wee, bee, bue)


def _node_final(nnf, enf2, Wt, Wb, bun):
    def body(nnf_ref, enf_ref, Wt_ref, Wb_ref, b_ref, out_ref):
        e = enf_ref[:V, :] + enf_ref[VP:VP + V, :]
        acc = jnp.dot(nnf_ref[...], Wt_ref[...], preferred_element_type=jnp.float32)
        acc += jnp.dot(e, Wb_ref[...], preferred_element_type=jnp.float32)
        out_ref[...] = jnp.maximum(acc + b_ref[...], 0.0)

    return pl.pallas_call(
        body,
        out_shape=jax.ShapeDtypeStruct((V, HP), jnp.float32),
    )(nnf, enf2, Wt, Wb, bun)


def _pad_w(W, rows, cols=HP):
    out = jnp.zeros((rows, cols), jnp.float32)
    return out.at[:W.shape[0], :W.shape[1]].set(W)


def _pad_b(b):
    return jnp.zeros((1, HP), jnp.float32).at[0, :b.shape[0]].set(b)


def kernel(node_feats, edge_feats, edge_index, W_nn, b_nn, W_en, b_en,
           W_un, b_un, W_l, b_l, W_r, b_r, W_ee, b_ee, W_ue, b_ue):
    src = edge_index[0].astype(jnp.int32).reshape(NW, NCHUNK, CH)
    dst = edge_index[1].astype(jnp.int32).reshape(NW, NCHUNK, CH)

    Wl = _pad_w(W_l, NIF)
    Wr = _pad_w(W_r, NIF)
    Wnn = _pad_w(W_nn, NIF)
    Wen = _pad_w(W_en, EIF)
    Wee = _pad_w(W_ee, EIF)
    W1 = _pad_w(W_ue[:H], HP)
    W2 = _pad_w(W_ue[H:2 * H], HP)
    W3 = _pad_w(W_ue[2 * H:], HP)
    Wt = _pad_w(W_un[:H], HP)
    Wb = _pad_w(W_un[H:], HP)

    T, nnf = _node_prep(node_feats, Wl, _pad_b(b_l), Wr, _pad_b(b_r),
                        Wnn, _pad_b(b_nn))
    e2n = _edge_prep(edge_feats, Wen, _pad_b(b_en))
    zrows = jnp.zeros((VP, HP), jnp.float32)
    gs, gd, enf2 = _sc_gather_scatter(T, src, dst, e2n, zrows)
    new_edge = _edge_final(gs, gd, edge_feats, W1, W2, W3, Wee,
                           _pad_b(b_ee), _pad_b(b_ue))
    new_node = _node_final(nnf, enf2, Wt, Wb, _pad_b(b_un))
    return new_node[:, :H], new_edge[:, :H]


# speedup vs baseline: 4.3105x; 1.0032x over previous
import functools

import jax
import jax.numpy as jnp
from jax import lax
from jax.experimental import pallas as pl
from jax.experimental.pallas import tpu as pltpu
from jax.experimental.pallas import tpu_sc as plsc

V = 10000
E = 320000
NIF = 128
EIF = 16
H = 50
HP = 64
NC = 2
NS = 16
NW = NC * NS
EW = E // NW
CH = 40
NCHUNK = EW // CH
VP = 10240
VSUB = VP // NS


def _node_prep(nf, Wl, bl, Wr, br, Wnn, bnn):
    def body(nf_ref, Wl_ref, bl_ref, Wr_ref, br_ref, Wnn_ref, bnn_ref,
             T_ref, nnf_ref):
        x = nf_ref[...]
        left = jnp.dot(x, Wl_ref[...], preferred_element_type=jnp.float32) + bl_ref[...]
        right = jnp.dot(x, Wr_ref[...], preferred_element_type=jnp.float32) + br_ref[...]
        T_ref[...] = jnp.concatenate([left, right], axis=1)
        nnf = jnp.dot(x, Wnn_ref[...], preferred_element_type=jnp.float32) + bnn_ref[...]
        nnf_ref[...] = jnp.maximum(nnf, 0.0)

    return pl.pallas_call(
        body,
        out_shape=[jax.ShapeDtypeStruct((V, 2 * HP), jnp.float32),
                   jax.ShapeDtypeStruct((V, HP), jnp.float32)],
    )(nf, Wl, bl, Wr, br, Wnn, bnn)


def _edge_prep(ef, Wen, ben):
    TE = 4000

    def body(ef_ref, W_ref, b_ref, out_ref):
        y = jnp.dot(ef_ref[...], W_ref[...], preferred_element_type=jnp.float32)
        out_ref[...] = jnp.maximum(y + b_ref[...], 0.0)

    return pl.pallas_call(
        body,
        grid=(E // TE,),
        in_specs=[pl.BlockSpec((TE, EIF), lambda i: (i, 0)),
                  pl.BlockSpec((EIF, HP), lambda i: (0, 0)),
                  pl.BlockSpec((1, HP), lambda i: (0, 0))],
        out_specs=pl.BlockSpec((TE, HP), lambda i: (i, 0)),
        out_shape=jax.ShapeDtypeStruct((E, HP), jnp.float32),
    )(ef, Wen, ben)


def _sc_gather_scatter(T, src3, dst3, e2n, zrows):
    mesh = plsc.VectorSubcoreMesh(core_axis_name="c", subcore_axis_name="s")

    @functools.partial(
        pl.kernel,
        mesh=mesh,
        compiler_params=pltpu.CompilerParams(use_tc_tiling_on_sc=True),
        out_type=[jax.ShapeDtypeStruct((E, 2 * HP), jnp.float32),
                  jax.ShapeDtypeStruct((E, 2 * HP), jnp.float32),
                  jax.ShapeDtypeStruct((2 * VP, HP), jnp.float32)],
        scratch_types=[pltpu.VMEM((CH,), jnp.int32),
                       pltpu.VMEM((CH,), jnp.int32),
                       pltpu.VMEM((CH,), jnp.int32),
                       pltpu.VMEM((CH,), jnp.int32),
                       pltpu.VMEM((CH,), jnp.int32),
                       pltpu.VMEM((CH,), jnp.int32),
                       pltpu.VMEM((CH,), jnp.int32),
                       pltpu.VMEM((CH,), jnp.int32),
                       pltpu.VMEM((CH, 2 * HP), jnp.float32),
                       pltpu.VMEM((CH, 2 * HP), jnp.float32),
                       pltpu.VMEM((CH, 2 * HP), jnp.float32),
                       pltpu.VMEM((CH, 2 * HP), jnp.float32),
                       pltpu.VMEM((CH, HP), jnp.float32),
                       pltpu.VMEM((CH, HP), jnp.float32),
                       pltpu.VMEM_SHARED((VP, HP), jnp.float32),
                       pltpu.SemaphoreType.DMA,
                       pltpu.SemaphoreType.DMA,
                       pltpu.SemaphoreType.DMA,
                       pltpu.SemaphoreType.DMA,
                       pltpu.SemaphoreType.DMA,
                       pltpu.SemaphoreType.DMA,
                       pltpu.SemaphoreType.DMA,
                       pltpu.SemaphoreType.DMA],
    )
    def k(T_hbm, src_hbm, dst_hbm, e2n_hbm, z_hbm, gs_out, gd_out, enf_out,
          si0, si1, si2, si3, di0, di1, di2, di3,
          gs_v0, gs_v1, gd_v0, gd_v1, ev0, ev1, acc,
          gsem0, gsem1, wsem0, wsem1, isem0, isem1, isem2, isem3):
        cid = lax.axis_index("c")
        sid = lax.axis_index("s")
        wid = sid * NC + cid
        sis = (si0, si1, si2, si3)
        dis = (di0, di1, di2, di3)
        gs_v = (gs_v0, gs_v1)
        gd_v = (gd_v0, gd_v1)
        ev = (ev0, ev1)
        gsems = (gsem0, gsem1)
        wsems = (wsem0, wsem1)
        isems = (isem0, isem1, isem2, isem3)
        pltpu.sync_copy(z_hbm.at[pl.ds(sid * VSUB, VSUB)],
                        acc.at[pl.ds(sid * VSUB, VSUB)])
        plsc.subcore_barrier()

        def i_start(i, s):
            pltpu.async_copy(src_hbm.at[wid, i], sis[s], isems[s])
            pltpu.async_copy(dst_hbm.at[wid, i], dis[s], isems[s])

        def i_wait(i, s):
            pltpu.make_async_copy(src_hbm.at[wid, i], sis[s], isems[s]).wait()
            pltpu.make_async_copy(dst_hbm.at[wid, i], dis[s], isems[s]).wait()

        def g_start(i, s, b):
            base = wid * EW + i * CH
            pltpu.async_copy(T_hbm.at[sis[s]], gs_v[b], gsems[b])
            pltpu.async_copy(T_hbm.at[dis[s]], gd_v[b], gsems[b])
            pltpu.async_copy(e2n_hbm.at[pl.ds(base, CH)], ev[b], gsems[b])

        def g_wait(i, s, b):
            base = wid * EW + i * CH
            pltpu.make_async_copy(T_hbm.at[sis[s]], gs_v[b], gsems[b]).wait()
            pltpu.make_async_copy(T_hbm.at[dis[s]], gd_v[b], gsems[b]).wait()
            pltpu.make_async_copy(e2n_hbm.at[pl.ds(base, CH)], ev[b], gsems[b]).wait()

        def w_start(i, s, b):
            base = wid * EW + i * CH
            pltpu.async_copy(gs_v[b], gs_out.at[pl.ds(base, CH)], wsems[b])
            pltpu.async_copy(gd_v[b], gd_out.at[pl.ds(base, CH)], wsems[b])
            pltpu.sync_copy(ev[b], acc.at[dis[s]], add=True)

        def w_wait(i, b):
            base = wid * EW + i * CH
            pltpu.make_async_copy(gs_v[b], gs_out.at[pl.ds(base, CH)], wsems[b]).wait()
            pltpu.make_async_copy(gd_v[b], gd_out.at[pl.ds(base, CH)], wsems[b]).wait()

        for s in range(4):
            i_start(s, s)
        i_wait(0, 0)
        i_wait(1, 1)
        g_start(0, 0, 0)
        g_start(1, 1, 1)

        def body(g, carry):
            for b4 in range(4):
                j = 4 * g + b4
                b = b4 % 2
                g_wait(j, b4, b)
                w_start(j, b4, b)
                w_wait(j, b)
                i_wait(j + 2, (b4 + 2) % 4)
                g_start(j + 2, (b4 + 2) % 4, b)
                i_start(j + 4, b4)
            return carry

        lax.fori_loop(0, (NCHUNK - 2) // 4, body, 0)
        for t in range(2):
            j = NCHUNK - 2 + t
            s = j % 4
            b = j % 2
            g_wait(j, s, b)
            w_start(j, s, b)
            w_wait(j, b)
        i_wait(NCHUNK, NCHUNK % 4)
        i_wait(NCHUNK + 1, (NCHUNK + 1) % 4)
        plsc.subcore_barrier()
        pltpu.sync_copy(acc.at[pl.ds(sid * VSUB, VSUB)],
                        enf_out.at[pl.ds(cid * VP + sid * VSUB, VSUB)])

    return k(T, src3, dst3, e2n, zrows)


def _edge_final(gs, gd, ef, W1, W2, W3, Wee, bee, bue):
    TE = 4000

    def body(gs_ref, gd_ref, ef_ref, W1_ref, W2_ref, W3_ref, Wee_ref,
             bee_ref, bue_ref, out_ref):
        g_s = gs_ref[...]
        g_d = gd_ref[...]
        first = jnp.maximum(g_s[:, :HP] + g_d[:, HP:], 0.0)
        second = jnp.maximum(g_s[:, HP:] + g_d[:, :HP], 0.0)
        third = jnp.dot(ef_ref[...], Wee_ref[...], preferred_element_type=jnp.float32)
        third = jnp.maximum(third + bee_ref[...], 0.0)
        acc = jnp.dot(first, W1_ref[...], preferred_element_type=jnp.float32)
        acc += jnp.dot(second, W2_ref[...], preferred_element_type=jnp.float32)
        acc += jnp.dot(third, W3_ref[...], preferred_element_type=jnp.float32)
        out_ref[...] = jnp.maximum(acc + bue_ref[...], 0.0)

    return pl.pallas_call(
        body,
        grid=(E // TE,),
        in_specs=[pl.BlockSpec((TE, 2 * HP), lambda i: (i, 0)),
                  pl.BlockSpec((TE, 2 * HP), lambda i: (i, 0)),
                  pl.BlockSpec((TE, EIF), lambda i: (i, 0)),
                  pl.BlockSpec((HP, HP), lambda i: (0, 0)),
                  pl.BlockSpec((HP, HP), lambda i: (0, 0)),
                  pl.BlockSpec((HP, HP), lambda i: (0, 0)),
                  pl.BlockSpec((EIF, HP), lambda i: (0, 0)),
                  pl.BlockSpec((1, HP), lambda i: (0, 0)),
                  pl.BlockSpec((1, HP), lambda i: (0, 0))],
        out_specs=pl.BlockSpec((TE, HP), lambda i: (i, 0)),
        out_shape=jax.ShapeDtypeStruct((E, HP), jnp.float32),
    )(gs, gd, ef, W1, W2, W3, Wee, bee, bue)


def _node_final(nnf, enf2, Wt, Wb, bun):
    def body(nnf_ref, enf_ref, Wt_ref, Wb_ref, b_ref, out_ref):
        e = enf_ref[:V, :] + enf_ref[VP:VP + V, :]
        acc = jnp.dot(nnf_ref[...], Wt_ref[...], preferred_element_type=jnp.float32)
        acc += jnp.dot(e, Wb_ref[...], preferred_element_type=jnp.float32)
        out_ref[...] = jnp.maximum(acc + b_ref[...], 0.0)

    return pl.pallas_call(
        body,
        out_shape=jax.ShapeDtypeStruct((V, HP), jnp.float32),
    )(nnf, enf2, Wt, Wb, bun)


def _pad_w(W, rows, cols=HP):
    out = jnp.zeros((rows, cols), jnp.float32)
    return out.at[:W.shape[0], :W.shape[1]].set(W)


def _pad_b(b):
    return jnp.zeros((1, HP), jnp.float32).at[0, :b.shape[0]].set(b)


def kernel(node_feats, edge_feats, edge_index, W_nn, b_nn, W_en, b_en,
           W_un, b_un, W_l, b_l, W_r, b_r, W_ee, b_ee, W_ue, b_ue):
    src = edge_index[0].astype(jnp.int32).reshape(NW, NCHUNK, CH)
    dst = edge_index[1].astype(jnp.int32).reshape(NW, NCHUNK, CH)

    Wl = _pad_w(W_l, NIF)
    Wr = _pad_w(W_r, NIF)
    Wnn = _pad_w(W_nn, NIF)
    Wen = _pad_w(W_en, EIF)
    Wee = _pad_w(W_ee, EIF)
    W1 = _pad_w(W_ue[:H], HP)
    W2 = _pad_w(W_ue[H:2 * H], HP)
    W3 = _pad_w(W_ue[2 * H:], HP)
    Wt = _pad_w(W_un[:H], HP)
    Wb = _pad_w(W_un[H:], HP)

    T, nnf = _node_prep(node_feats, Wl, _pad_b(b_l), Wr, _pad_b(b_r),
                        Wnn, _pad_b(b_nn))
    e2n = _edge_prep(edge_feats, Wen, _pad_b(b_en))
    zrows = jnp.zeros((VP, HP), jnp.float32)
    gs, gd, enf2 = _sc_gather_scatter(T, src, dst, e2n, zrows)
    new_edge = _edge_final(gs, gd, edge_feats, W1, W2, W3, Wee,
                           _pad_b(b_ee), _pad_b(b_ue))
    new_node = _node_final(nnf, enf2, Wt, Wb, _pad_b(b_un))
    return new_node[:, :H], new_edge[:, :H]


# speedup vs baseline: 4.6241x; 1.0727x over previous
import functools

import jax
import jax.numpy as jnp
from jax import lax
from jax.experimental import pallas as pl
from jax.experimental.pallas import tpu as pltpu
from jax.experimental.pallas import tpu_sc as plsc

V = 10000
E = 320000
NIF = 128
EIF = 16
H = 50
HP = 64
NC = 2
NS = 16
NW = NC * NS
EW = E // NW
CH = 40
NCHUNK = EW // CH
VP = 10240
VSUB = VP // NS


def _node_prep(nf, Wl, bl, Wr, br, Wnn, bnn):
    def body(nf_ref, Wl_ref, bl_ref, Wr_ref, br_ref, Wnn_ref, bnn_ref,
             T_ref, nnf_ref):
        x = nf_ref[...]
        left = jnp.dot(x, Wl_ref[...], preferred_element_type=jnp.float32) + bl_ref[...]
        right = jnp.dot(x, Wr_ref[...], preferred_element_type=jnp.float32) + br_ref[...]
        T_ref[...] = jnp.concatenate([left, right], axis=1)
        nnf = jnp.dot(x, Wnn_ref[...], preferred_element_type=jnp.float32) + bnn_ref[...]
        nnf_ref[...] = jnp.maximum(nnf, 0.0)

    return pl.pallas_call(
        body,
        out_shape=[jax.ShapeDtypeStruct((V, 2 * HP), jnp.float32),
                   jax.ShapeDtypeStruct((V, HP), jnp.float32)],
    )(nf, Wl, bl, Wr, br, Wnn, bnn)


def _edge_prep(ef, Wen, ben):
    TE = 4000

    def body(ef_ref, W_ref, b_ref, out_ref):
        y = jnp.dot(ef_ref[...], W_ref[...], preferred_element_type=jnp.float32)
        out_ref[...] = jnp.maximum(y + b_ref[...], 0.0)

    return pl.pallas_call(
        body,
        grid=(E // TE,),
        in_specs=[pl.BlockSpec((TE, EIF), lambda i: (i, 0)),
                  pl.BlockSpec((EIF, HP), lambda i: (0, 0)),
                  pl.BlockSpec((1, HP), lambda i: (0, 0))],
        out_specs=pl.BlockSpec((TE, HP), lambda i: (i, 0)),
        out_shape=jax.ShapeDtypeStruct((E, HP), jnp.float32),
    )(ef, Wen, ben)


def _sc_gather_scatter(T, src3, dst3, e2n, zrows):
    mesh = plsc.VectorSubcoreMesh(core_axis_name="c", subcore_axis_name="s")

    @functools.partial(
        pl.kernel,
        mesh=mesh,
        out_type=[jax.ShapeDtypeStruct((E, 2 * HP), jnp.float32),
                  jax.ShapeDtypeStruct((2 * VP, HP), jnp.float32)],
        scratch_types=[pltpu.VMEM((CH,), jnp.int32),
                       pltpu.VMEM((CH,), jnp.int32),
                       pltpu.VMEM((CH,), jnp.int32),
                       pltpu.VMEM((CH,), jnp.int32),
                       pltpu.VMEM((CH,), jnp.int32),
                       pltpu.VMEM((CH,), jnp.int32),
                       pltpu.VMEM((CH,), jnp.int32),
                       pltpu.VMEM((CH,), jnp.int32),
                       pltpu.VMEM((CH, 2 * HP), jnp.float32),
                       pltpu.VMEM((CH, 2 * HP), jnp.float32),
                       pltpu.VMEM((CH, 2 * HP), jnp.float32),
                       pltpu.VMEM((CH, 2 * HP), jnp.float32),
                       pltpu.VMEM((CH, HP), jnp.float32),
                       pltpu.VMEM((CH, HP), jnp.float32),
                       pltpu.VMEM_SHARED((VP, HP), jnp.float32),
                       pltpu.SemaphoreType.DMA,
                       pltpu.SemaphoreType.DMA,
                       pltpu.SemaphoreType.DMA,
                       pltpu.SemaphoreType.DMA,
                       pltpu.SemaphoreType.DMA,
                       pltpu.SemaphoreType.DMA,
                       pltpu.SemaphoreType.DMA,
                       pltpu.SemaphoreType.DMA],
    )
    def k(T_hbm, src_hbm, dst_hbm, e2n_hbm, z_hbm, fo_out, enf_out,
          si0, si1, si2, si3, di0, di1, di2, di3,
          gs_v0, gs_v1, gd_v0, gd_v1, ev0, ev1, acc,
          gsem0, gsem1, wsem0, wsem1, isem0, isem1, isem2, isem3):
        cid = lax.axis_index("c")
        sid = lax.axis_index("s")
        wid = sid * NC + cid
        sis = (si0, si1, si2, si3)
        dis = (di0, di1, di2, di3)
        gs_v = (gs_v0, gs_v1)
        gd_v = (gd_v0, gd_v1)
        ev = (ev0, ev1)
        gsems = (gsem0, gsem1)
        wsems = (wsem0, wsem1)
        isems = (isem0, isem1, isem2, isem3)
        pltpu.sync_copy(z_hbm.at[pl.ds(sid * VSUB, VSUB)],
                        acc.at[pl.ds(sid * VSUB, VSUB)])
        plsc.subcore_barrier()

        def i_start(i, s):
            pltpu.async_copy(src_hbm.at[wid, i], sis[s], isems[s])
            pltpu.async_copy(dst_hbm.at[wid, i], dis[s], isems[s])

        def i_wait(i, s):
            pltpu.make_async_copy(src_hbm.at[wid, i], sis[s], isems[s]).wait()
            pltpu.make_async_copy(dst_hbm.at[wid, i], dis[s], isems[s]).wait()

        def g_start(i, s, b):
            base = wid * EW + i * CH
            pltpu.async_copy(T_hbm.at[sis[s]], gs_v[b], gsems[b])
            pltpu.async_copy(T_hbm.at[dis[s]], gd_v[b], gsems[b])
            pltpu.async_copy(e2n_hbm.at[pl.ds(base, CH)], ev[b], gsems[b])

        def g_wait(i, s, b):
            base = wid * EW + i * CH
            pltpu.make_async_copy(T_hbm.at[sis[s]], gs_v[b], gsems[b]).wait()
            pltpu.make_async_copy(T_hbm.at[dis[s]], gd_v[b], gsems[b]).wait()
            pltpu.make_async_copy(e2n_hbm.at[pl.ds(base, CH)], ev[b], gsems[b]).wait()

        def fuse(b):
            gvb = gs_v[b]
            gdb = gd_v[b]

            def row(r, carry):
                for kk in range(HP // 16):
                    ls = gvb[r, pl.ds(16 * kk, 16)]
                    rs = gvb[r, pl.ds(HP + 16 * kk, 16)]
                    ld = gdb[r, pl.ds(16 * kk, 16)]
                    rd = gdb[r, pl.ds(HP + 16 * kk, 16)]
                    gvb[r, pl.ds(16 * kk, 16)] = jnp.maximum(ls + rd, 0.0)
                    gvb[r, pl.ds(HP + 16 * kk, 16)] = jnp.maximum(rs + ld, 0.0)
                return carry

            lax.fori_loop(0, CH, row, 0)

        def w_start(i, s, b):
            base = wid * EW + i * CH
            pltpu.async_copy(gs_v[b], fo_out.at[pl.ds(base, CH)], wsems[b])
            pltpu.sync_copy(ev[b], acc.at[dis[s]], add=True)

        def w_wait(i, b):
            base = wid * EW + i * CH
            pltpu.make_async_copy(gs_v[b], fo_out.at[pl.ds(base, CH)], wsems[b]).wait()

        for s in range(4):
            i_start(s, s)
        i_wait(0, 0)
        i_wait(1, 1)
        g_start(0, 0, 0)
        g_start(1, 1, 1)

        def body(g, carry):
            for b4 in range(4):
                j = 4 * g + b4
                b = b4 % 2
                g_wait(j, b4, b)
                fuse(b)
                w_start(j, b4, b)
                w_wait(j, b)
                i_wait(j + 2, (b4 + 2) % 4)
                g_start(j + 2, (b4 + 2) % 4, b)
                i_start(j + 4, b4)
            return carry

        lax.fori_loop(0, (NCHUNK - 2) // 4, body, 0)
        for t in range(2):
            j = NCHUNK - 2 + t
            s = j % 4
            b = j % 2
            g_wait(j, s, b)
            fuse(b)
            w_start(j, s, b)
            w_wait(j, b)
        i_wait(NCHUNK, NCHUNK % 4)
        i_wait(NCHUNK + 1, (NCHUNK + 1) % 4)
        plsc.subcore_barrier()
        pltpu.sync_copy(acc.at[pl.ds(sid * VSUB, VSUB)],
                        enf_out.at[pl.ds(cid * VP + sid * VSUB, VSUB)])

    return k(T, src3, dst3, e2n, zrows)


def _edge_final(fo, ef, W1, W2, W3, Wee, bee, bue):
    TE = 4000

    def body(fo_ref, ef_ref, W1_ref, W2_ref, W3_ref, Wee_ref,
             bee_ref, bue_ref, out_ref):
        f_o = fo_ref[...]
        first = f_o[:, :HP]
        second = f_o[:, HP:]
        third = jnp.dot(ef_ref[...], Wee_ref[...], preferred_element_type=jnp.float32)
        third = jnp.maximum(third + bee_ref[...], 0.0)
        acc = jnp.dot(first, W1_ref[...], preferred_element_type=jnp.float32)
        acc += jnp.dot(second, W2_ref[...], preferred_element_type=jnp.float32)
        acc += jnp.dot(third, W3_ref[...], preferred_element_type=jnp.float32)
        out_ref[...] = jnp.maximum(acc + bue_ref[...], 0.0)

    return pl.pallas_call(
        body,
        grid=(E // TE,),
        in_specs=[pl.BlockSpec((TE, 2 * HP), lambda i: (i, 0)),
                  pl.BlockSpec((TE, EIF), lambda i: (i, 0)),
                  pl.BlockSpec((HP, HP), lambda i: (0, 0)),
                  pl.BlockSpec((HP, HP), lambda i: (0, 0)),
                  pl.BlockSpec((HP, HP), lambda i: (0, 0)),
                  pl.BlockSpec((EIF, HP), lambda i: (0, 0)),
                  pl.BlockSpec((1, HP), lambda i: (0, 0)),
                  pl.BlockSpec((1, HP), lambda i: (0, 0))],
        out_specs=pl.BlockSpec((TE, HP), lambda i: (i, 0)),
        out_shape=jax.ShapeDtypeStruct((E, HP), jnp.float32),
    )(fo, ef, W1, W2, W3, Wee, bee, bue)


def _node_final(nnf, enf2, Wt, Wb, bun):
    def body(nnf_ref, enf_ref, Wt_ref, Wb_ref, b_ref, out_ref):
        e = enf_ref[:V, :] + enf_ref[VP:VP + V, :]
        acc = jnp.dot(nnf_ref[...], Wt_ref[...], preferred_element_type=jnp.float32)
        acc += jnp.dot(e, Wb_ref[...], preferred_element_type=jnp.float32)
        out_ref[...] = jnp.maximum(acc + b_ref[...], 0.0)

    return pl.pallas_call(
        body,
        out_shape=jax.ShapeDtypeStruct((V, HP), jnp.float32),
    )(nnf, enf2, Wt, Wb, bun)


def _pad_w(W, rows, cols=HP):
    out = jnp.zeros((rows, cols), jnp.float32)
    return out.at[:W.shape[0], :W.shape[1]].set(W)


def _pad_b(b):
    return jnp.zeros((1, HP), jnp.float32).at[0, :b.shape[0]].set(b)


def kernel(node_feats, edge_feats, edge_index, W_nn, b_nn, W_en, b_en,
           W_un, b_un, W_l, b_l, W_r, b_r, W_ee, b_ee, W_ue, b_ue):
    src = edge_index[0].astype(jnp.int32).reshape(NW, NCHUNK, CH)
    dst = edge_index[1].astype(jnp.int32).reshape(NW, NCHUNK, CH)

    Wl = _pad_w(W_l, NIF)
    Wr = _pad_w(W_r, NIF)
    Wnn = _pad_w(W_nn, NIF)
    Wen = _pad_w(W_en, EIF)
    Wee = _pad_w(W_ee, EIF)
    W1 = _pad_w(W_ue[:H], HP)
    W2 = _pad_w(W_ue[H:2 * H], HP)
    W3 = _pad_w(W_ue[2 * H:], HP)
    Wt = _pad_w(W_un[:H], HP)
    Wb = _pad_w(W_un[H:], HP)

    T, nnf = _node_prep(node_feats, Wl, _pad_b(b_l), Wr, _pad_b(b_r),
                        Wnn, _pad_b(b_nn))
    e2n = _edge_prep(edge_feats, Wen, _pad_b(b_en))
    zrows = jnp.zeros((VP, HP), jnp.float32)
    fo, enf2 = _sc_gather_scatter(T, src, dst, e2n, zrows)
    new_edge = _edge_final(fo, edge_feats, W1, W2, W3, Wee,
                           _pad_b(b_ee), _pad_b(b_ue))
    new_node = _node_final(nnf, enf2, Wt, Wb, _pad_b(b_un))
    return new_node[:, :H], new_edge[:, :H]
